# Initial kernel scaffold; baseline (speedup 1.0000x reference)
import jax
import jax.numpy as jnp
from jax.experimental import pallas as pl


def kernel(user_emb, spot_emb, W_user, b_user, W_spot, b_spot, src_user_to_spot, dst_user_to_spot, src_spot_to_user, dst_spot_to_user):
    raise NotImplementedError("write your pallas kernel here")


# speedup vs baseline: 5.3245x; 5.3245x over previous
import functools

import jax
import jax.numpy as jnp
from jax import lax
from jax.experimental import pallas as pl
from jax.experimental.pallas import tpu as pltpu
from jax.experimental.pallas import tpu_sc as plsc

D = 128
LANES = 128
NCORES = 2
NSUB = 16
NTILES = NCORES * NSUB


def _mesh():
    return plsc.VectorSubcoreMesh(core_axis_name="c", subcore_axis_name="s")


def _split(total, parts, idx):
    q, r = divmod(total, parts)
    n = q + jnp.where(idx < r, 1, 0)
    start = q * idx + jnp.minimum(idx, r)
    return start, n


def _make_bincounts(R, CU, CS):
    CH_U = CU // NSUB
    CH_S = CS // NSUB

    def body(src_us, dst_us, src_su, dst_su, z1d,
             out_a, out_b, out_c, out_d,
             cnt_a, cnt_b, cnt_c, cnt_d, ones_v, idx_v, stage, sem):
        c = lax.axis_index("c")
        s = lax.axis_index("s")
        wid = c * NSUB + s

        for k in range(LANES // 16):
            ones_v[pl.ds(k * 16, 16)] = jnp.full((16,), 1.0, jnp.float32)

        pltpu.sync_copy(z1d, stage)
        for cnt, ch in ((cnt_a, CH_U), (cnt_b, CH_S),
                        (cnt_c, CH_S), (cnt_d, CH_U)):
            pltpu.sync_copy(stage.at[pl.ds(0, ch)], cnt.at[pl.ds(s * ch, ch)])
        plsc.subcore_barrier()

        start, n = _split(R, NTILES, wid)

        def scan(idx1d, cnt):
            def step(i, carry):
                j = start + i
                pltpu.sync_copy(idx1d.at[pl.ds(j * LANES, LANES)], idx_v)
                pltpu.sync_copy(ones_v, cnt.at[idx_v], add=True)
                return carry
            lax.fori_loop(0, n, step, 0)

        scan(src_us, cnt_a)
        scan(dst_us, cnt_b)
        scan(src_su, cnt_c)
        scan(dst_su, cnt_d)
        plsc.subcore_barrier()

        for cnt, out, n in ((cnt_a, out_a, CU), (cnt_b, out_b, CS),
                            (cnt_c, out_c, CS), (cnt_d, out_d, CU)):
            ch = n // NSUB
            pltpu.sync_copy(cnt.at[pl.ds(s * ch, ch)], stage.at[pl.ds(0, ch)])
            pltpu.sync_copy(stage.at[pl.ds(0, ch)],
                            out.at[pl.ds(c * n + s * ch, ch)])

    return pl.kernel(
        body,
        out_type=(
            jax.ShapeDtypeStruct((NCORES * CU,), jnp.float32),
            jax.ShapeDtypeStruct((NCORES * CS,), jnp.float32),
            jax.ShapeDtypeStruct((NCORES * CS,), jnp.float32),
            jax.ShapeDtypeStruct((NCORES * CU,), jnp.float32),
        ),
        mesh=_mesh(),
        scratch_types=[
            pltpu.VMEM_SHARED((CU,), jnp.float32),
            pltpu.VMEM_SHARED((CS,), jnp.float32),
            pltpu.VMEM_SHARED((CS,), jnp.float32),
            pltpu.VMEM_SHARED((CU,), jnp.float32),
            pltpu.VMEM((LANES,), jnp.float32),
            pltpu.VMEM((LANES,), jnp.int32),
            pltpu.VMEM((CH_U,), jnp.float32),
            pltpu.SemaphoreType.DMA,
        ],
    )


def _make_scatter_us(R, CS):
    CH = CS // NSUB

    def body(y, src, dst, z2d, out, acc, sidx, didx, rows, sem):
        c = lax.axis_index("c")
        s = lax.axis_index("s")
        wid = c * NSUB + s

        pltpu.sync_copy(z2d, rows)
        for t in range(CH // LANES):
            pltpu.sync_copy(rows, acc.at[pl.ds(s * CH + t * LANES, LANES)])
        plsc.subcore_barrier()

        start, n = _split(R, NTILES, wid)

        def step(i, carry):
            j = start + i
            pltpu.sync_copy(src.at[pl.ds(j * LANES, LANES)], sidx)
            pltpu.sync_copy(dst.at[pl.ds(j * LANES, LANES)], didx)
            pltpu.async_copy(y.at[sidx], rows, sem).wait()
            pltpu.sync_copy(rows, acc.at[didx], add=True)
            return carry

        lax.fori_loop(0, n, step, 0)
        plsc.subcore_barrier()

        for t in range(CH // LANES):
            o = s * CH + t * LANES
            pltpu.sync_copy(acc.at[pl.ds(o, LANES)], rows)
            pltpu.sync_copy(rows, out.at[c, pl.ds(o, LANES)])

    return pl.kernel(
        body,
        out_type=jax.ShapeDtypeStruct((NCORES, CS, D), jnp.float32),
        mesh=_mesh(),
        scratch_types=[
            pltpu.VMEM_SHARED((CS, D), jnp.float32),
            pltpu.VMEM((LANES,), jnp.int32),
            pltpu.VMEM((LANES,), jnp.int32),
            pltpu.VMEM((LANES, D), jnp.float32),
            pltpu.SemaphoreType.DMA,
        ],
    )


def _make_scatter_su(R, CU, NR):
    ACC = NR + 128
    CH = ACC // NSUB
    CHO = NR // NSUB

    def _chunks(total):
        full, tail = divmod(total, LANES)
        return [LANES] * full + ([tail] if tail else [])

    def body(y, src, dst, z2d, out, acc, sidx, didx, aidx, rows, sem):
        c = lax.axis_index("c")
        s = lax.axis_index("s")

        start, n = _split(R, NSUB, s)

        for r in range(2):
            base = (2 * c + r) * NR

            if r == 0:
                pltpu.sync_copy(z2d, rows)
            o = 0
            for w in _chunks(CH):
                pltpu.sync_copy(rows.at[pl.ds(0, w)],
                                acc.at[pl.ds(s * CH + o, w)])
                o += w
            plsc.subcore_barrier()

            def step(i, carry):
                j = start + i
                pltpu.sync_copy(src.at[pl.ds(j * LANES, LANES)], sidx)
                pltpu.sync_copy(dst.at[pl.ds(j * LANES, LANES)], didx)
                for k in range(LANES // 16):
                    dv = didx[pl.ds(k * 16, 16)]
                    local = dv - base
                    ok = (local >= 0) & (local < NR)
                    aidx[pl.ds(k * 16, 16)] = jnp.where(ok, local, NR)
                pltpu.async_copy(y.at[sidx], rows, sem).wait()
                pltpu.sync_copy(rows, acc.at[aidx], add=True)
                return carry

            lax.fori_loop(0, n, step, 0)
            plsc.subcore_barrier()

            o = 0
            for w in _chunks(CHO):
                pltpu.sync_copy(acc.at[pl.ds(s * CHO + o, w)],
                                rows.at[pl.ds(0, w)])
                pltpu.sync_copy(rows.at[pl.ds(0, w)],
                                out.at[pl.ds(base + s * CHO + o, w)])
                o += w
            if r == 0:
                pltpu.sync_copy(z2d, rows)
            plsc.subcore_barrier()

    return pl.kernel(
        body,
        out_type=jax.ShapeDtypeStruct((CU, D), jnp.float32),
        mesh=_mesh(),
        scratch_types=[
            pltpu.VMEM_SHARED((ACC, D), jnp.float32),
            pltpu.VMEM((LANES,), jnp.int32),
            pltpu.VMEM((LANES,), jnp.int32),
            pltpu.VMEM((LANES,), jnp.int32),
            pltpu.VMEM((LANES, D), jnp.float32),
            pltpu.SemaphoreType.DMA,
        ],
    )


def _rsqrt_factors(cnt_a, cnt_b, cnt_c, cnt_d):
    CU = cnt_a.shape[0] // NCORES
    CS = cnt_b.shape[0] // NCORES

    def body(a, b, c, d, fa, fb, fc, fd):
        for x, f, n in ((a, fa, CU), (b, fb, CS), (c, fc, CS), (d, fd, CU)):
            sm = x[pl.ds(0, n)] + x[pl.ds(n, n)]
            sm = jnp.where(sm == 0.0, 1e-6, sm)
            f[:] = lax.rsqrt(sm)

    return pl.pallas_call(
        body,
        out_shape=(
            jax.ShapeDtypeStruct((CU,), jnp.float32),
            jax.ShapeDtypeStruct((CS,), jnp.float32),
            jax.ShapeDtypeStruct((CS,), jnp.float32),
            jax.ShapeDtypeStruct((CU,), jnp.float32),
        ),
    )(cnt_a, cnt_b, cnt_c, cnt_d)


_BLK = 1024


def _row_scale(x, f):
    N = x.shape[0]

    def body(x_ref, f_ref, o_ref):
        o_ref[:, :] = x_ref[:, :] * f_ref[:][:, None]

    return pl.pallas_call(
        body,
        grid=(N // _BLK,),
        in_specs=[
            pl.BlockSpec((_BLK, D), lambda i: (i, 0)),
            pl.BlockSpec((_BLK,), lambda i: (i,)),
        ],
        out_specs=pl.BlockSpec((_BLK, D), lambda i: (i, 0)),
        out_shape=jax.ShapeDtypeStruct((N, D), jnp.float32),
    )(x, f)


def _post_partial(z2, fpost, fnext):
    N = z2.shape[1]

    def body(z_ref, fp_ref, fn_ref, act_ref, nxt_ref):
        zz = z_ref[0] + z_ref[1]
        act = jnp.maximum(zz * fp_ref[:][:, None], 0.0)
        act_ref[:, :] = act
        nxt_ref[:, :] = act * fn_ref[:][:, None]

    return pl.pallas_call(
        body,
        grid=(N // _BLK,),
        in_specs=[
            pl.BlockSpec((NCORES, _BLK, D), lambda i: (0, i, 0)),
            pl.BlockSpec((_BLK,), lambda i: (i,)),
            pl.BlockSpec((_BLK,), lambda i: (i,)),
        ],
        out_specs=(
            pl.BlockSpec((_BLK, D), lambda i: (i, 0)),
            pl.BlockSpec((_BLK, D), lambda i: (i, 0)),
        ),
        out_shape=(
            jax.ShapeDtypeStruct((N, D), jnp.float32),
            jax.ShapeDtypeStruct((N, D), jnp.float32),
        ),
    )(z2, fpost, fnext)


def _post_full(z, fpost, fnext):
    N = z.shape[0]

    def body(z_ref, fp_ref, fn_ref, act_ref, nxt_ref):
        act = jnp.maximum(z_ref[:, :] * fp_ref[:][:, None], 0.0)
        act_ref[:, :] = act
        nxt_ref[:, :] = act * fn_ref[:][:, None]

    return pl.pallas_call(
        body,
        grid=(N // _BLK,),
        in_specs=[
            pl.BlockSpec((_BLK, D), lambda i: (i, 0)),
            pl.BlockSpec((_BLK,), lambda i: (i,)),
            pl.BlockSpec((_BLK,), lambda i: (i,)),
        ],
        out_specs=(
            pl.BlockSpec((_BLK, D), lambda i: (i, 0)),
            pl.BlockSpec((_BLK, D), lambda i: (i, 0)),
        ),
        out_shape=(
            jax.ShapeDtypeStruct((N, D), jnp.float32),
            jax.ShapeDtypeStruct((N, D), jnp.float32),
        ),
    )(z, fpost, fnext)


def _finalize(x0, act1, act2, w):
    N = x0.shape[0]

    def body(x0_ref, a1_ref, a2_ref, w_ref, mean_ref, proj_ref):
        m = (x0_ref[:, :] + a1_ref[:, :] + a2_ref[:, :]) * (1.0 / 3.0)
        mean_ref[:, :] = m
        proj_ref[:] = jnp.sum(m * w_ref[:][None, :], axis=1)

    return pl.pallas_call(
        body,
        grid=(N // _BLK,),
        in_specs=[
            pl.BlockSpec((_BLK, D), lambda i: (i, 0)),
            pl.BlockSpec((_BLK, D), lambda i: (i, 0)),
            pl.BlockSpec((_BLK, D), lambda i: (i, 0)),
            pl.BlockSpec((D,), lambda i: (0,)),
        ],
        out_specs=(
            pl.BlockSpec((_BLK, D), lambda i: (i, 0)),
            pl.BlockSpec((_BLK,), lambda i: (i,)),
        ),
        out_shape=(
            jax.ShapeDtypeStruct((N, D), jnp.float32),
            jax.ShapeDtypeStruct((N,), jnp.float32),
        ),
    )(x0, act1, act2, w)


def kernel(user_emb, spot_emb, W_user, b_user, W_spot, b_spot,
           src_user_to_spot, dst_user_to_spot,
           src_spot_to_user, dst_spot_to_user):
    NU, _ = user_emb.shape
    NS = spot_emb.shape[0]
    E = src_user_to_spot.shape[0]
    assert E % LANES == 0
    R = E // LANES

    NR = (-(-NU // 4) + 127) // 128 * 128
    CU = 4 * NR
    CS = -(-NS // _BLK) * _BLK

    src_us = src_user_to_spot.astype(jnp.int32)
    dst_us = dst_user_to_spot.astype(jnp.int32)
    src_su = src_spot_to_user.astype(jnp.int32)
    dst_su = dst_spot_to_user.astype(jnp.int32)

    z1d = jnp.zeros((CU // NSUB,), jnp.float32)
    z2d = jnp.zeros((LANES, D), jnp.float32)

    bincounts = _make_bincounts(R, CU, CS)
    scatter_us = _make_scatter_us(R, CS)
    scatter_su = _make_scatter_su(R, CU, NR)

    cnt_a, cnt_b, cnt_c, cnt_d = bincounts(src_us, dst_us, src_su, dst_su, z1d)
    fa, fb, fc, fd = _rsqrt_factors(cnt_a, cnt_b, cnt_c, cnt_d)

    x_user = jnp.pad(user_emb, ((0, CU - NU), (0, 0)))
    x_spot = jnp.pad(spot_emb, ((0, CS - NS), (0, 0)))

    y_user = _row_scale(x_user, fa)
    y_spot = _row_scale(x_spot, fc)
    zs_p = scatter_us(y_user, src_us, dst_us, z2d)
    zu = scatter_su(y_spot, src_su, dst_su, z2d)
    act_s1, y_spot1 = _post_partial(zs_p, fb, fc)
    act_u1, y_user1 = _post_full(zu, fd, fa)

    zs_p2 = scatter_us(y_user1, src_us, dst_us, z2d)
    zu2 = scatter_su(y_spot1, src_su, dst_su, z2d)
    act_s2, _ = _post_partial(zs_p2, fb, fc)
    act_u2, _ = _post_full(zu2, fd, fa)

    w_u = W_user.reshape(D)
    w_s = W_spot.reshape(D)
    mean_u, proj_u = _finalize(x_user, act_u1, act_u2, w_u)
    mean_s, proj_s = _finalize(x_spot, act_s1, act_s2, w_s)

    x_user_out = mean_u[:NU]
    x_spot_out = mean_s[:NS]
    out_user = proj_u[:NU, None] + b_user
    out_spot = proj_s[:NS, None] + b_spot
    return (x_user_out, x_spot_out, out_user, out_spot)
